# Initial kernel scaffold; baseline (speedup 1.0000x reference)
import jax
import jax.numpy as jnp
from jax.experimental import pallas as pl


def kernel(x, edge_index, batch, W1, b1, W2, b2, W3, b3, Wl, bl):
    raise NotImplementedError("write your pallas kernel here")


# speedup vs baseline: 3.6860x; 3.6860x over previous
import functools
import jax
import jax.numpy as jnp
from jax import lax
from jax.experimental import pallas as pl
from jax.experimental.pallas import tpu as pltpu
from jax.experimental.pallas import tpu_sc as plsc

N = 10000
E = 320000
D = 128
G = 128
DOUT = 10

NC = 2
NS = 16
NW = NC * NS
CHUNK = 128
NCH = 80
EPW = NCH * CHUNK
EP = EPW * NW
NPAD = 10240
RPT = NPAD // NS

BLK = 400
NBLK = N // BLK

_mesh = plsc.VectorSubcoreMesh(
    core_axis_name="c", subcore_axis_name="s", num_cores=NC, num_subcores=NS)


@functools.partial(
    pl.kernel,
    out_type=jax.ShapeDtypeStruct((NW, NPAD), jnp.float32),
    mesh=_mesh,
    scratch_types=[
        pltpu.VMEM((NCH, CHUNK), jnp.int32),
        pltpu.VMEM((NPAD,), jnp.float32),
    ],
    compiler_params=pltpu.CompilerParams(needs_layout_passes=False),
)
def _sc_degree(packed_hbm, out_hbm, pk_v, deg_v):
    cid = lax.axis_index("c")
    sid = lax.axis_index("s")
    wid = sid * NC + cid

    def zgrp(i, _):
        deg_v[pl.ds(i * 16, 16)] = jnp.zeros((16,), jnp.float32)
        return 0
    lax.fori_loop(jnp.int32(0), jnp.int32(NPAD // 16), zgrp, 0)

    pltpu.sync_copy(packed_hbm.at[wid], pk_v)
    ones = jnp.full((16,), 1.0, jnp.float32)

    def body(c, _):
        for j in range(CHUNK // 16):
            idx = pk_v[c, pl.ds(j * 16, 16)] >> 16
            plsc.addupdate_scatter(deg_v, [idx], ones)
        return 0
    lax.fori_loop(jnp.int32(0), jnp.int32(NCH), body, 0)

    pltpu.sync_copy(deg_v, out_hbm.at[wid])


SPL = 5200
AGGR = 5248
RPT2 = AGGR // NS


@functools.partial(
    pl.kernel,
    out_type=jax.ShapeDtypeStruct((NC, 2, AGGR, D), jnp.float32),
    mesh=_mesh,
    scratch_types=[
        pltpu.VMEM((NCH, CHUNK), jnp.int32),
        pltpu.VMEM((NCH, CHUNK), jnp.int32),
        pltpu.VMEM((CHUNK, D), jnp.float32),
        pltpu.VMEM((CHUNK, D), jnp.float32),
        pltpu.SemaphoreType.DMA,
        pltpu.SemaphoreType.DMA,
        pltpu.VMEM_SHARED((AGGR, D), jnp.float32),
    ],
    compiler_params=pltpu.CompilerParams(needs_layout_passes=False),
)
def _sc_edge_scatter(hd_hbm, packed_hbm, out_hbm,
                     src_v, dst_v, buf0, buf1, sem0, sem1, agg_sp):
    cid = lax.axis_index("c")
    sid = lax.axis_index("s")
    wid = sid * NC + cid

    pltpu.sync_copy(packed_hbm.at[wid], src_v)
    def unsrc(c, _):
        for j in range(CHUNK // 16):
            v = src_v[c, pl.ds(j * 16, 16)]
            src_v[c, pl.ds(j * 16, 16)] = v & jnp.int32(0xFFFF)
        return 0
    lax.fori_loop(jnp.int32(0), jnp.int32(NCH), unsrc, 0)

    for p in range(2):
        def zrow(r, _):
            for j in range(D // 16):
                buf0[r, pl.ds(j * 16, 16)] = jnp.zeros((16,), jnp.float32)
            return 0
        lax.fori_loop(jnp.int32(0), jnp.int32(CHUNK), zrow, 0)
        pltpu.sync_copy(buf0, agg_sp.at[pl.ds(sid * RPT2, CHUNK)])
        pltpu.sync_copy(buf0, agg_sp.at[pl.ds(sid * RPT2 + CHUNK, CHUNK)])
        pltpu.sync_copy(buf0.at[pl.ds(0, RPT2 - 2 * CHUNK)],
                        agg_sp.at[pl.ds(sid * RPT2 + 2 * CHUNK,
                                        RPT2 - 2 * CHUNK)])

        pltpu.sync_copy(packed_hbm.at[wid], dst_v)
        def undst(c, _):
            for j in range(CHUNK // 16):
                d = (dst_v[c, pl.ds(j * 16, 16)] >> 16) - jnp.int32(p * SPL)
                ok = (d >= 0) & (d < SPL)
                dst_v[c, pl.ds(j * 16, 16)] = jnp.where(ok, d, jnp.int32(SPL))
            return 0
        lax.fori_loop(jnp.int32(0), jnp.int32(NCH), undst, 0)
        plsc.subcore_barrier()

        pltpu.async_copy(hd_hbm.at[src_v.at[jnp.int32(0)]], buf0, sem0)
        pltpu.async_copy(hd_hbm.at[src_v.at[jnp.int32(1)]], buf1, sem1)

        def body(i, _):
            c0 = jnp.int32(2) * i
            pltpu.make_async_copy(hd_hbm.at[src_v.at[c0]], buf0, sem0).wait()
            pltpu.sync_copy(buf0, agg_sp.at[dst_v.at[c0]], add=True)

            @pl.when(c0 + 2 < NCH)
            def _():
                pltpu.async_copy(hd_hbm.at[src_v.at[c0 + 2]], buf0, sem0)

            c1 = c0 + 1
            pltpu.make_async_copy(hd_hbm.at[src_v.at[c1]], buf1, sem1).wait()
            pltpu.sync_copy(buf1, agg_sp.at[dst_v.at[c1]], add=True)

            @pl.when(c1 + 2 < NCH)
            def _():
                pltpu.async_copy(hd_hbm.at[src_v.at[c1 + 2]], buf1, sem1)
            return 0
        lax.fori_loop(jnp.int32(0), jnp.int32(NCH // 2), body, 0)

        plsc.subcore_barrier()
        pltpu.sync_copy(agg_sp.at[pl.ds(sid * RPT2, RPT2)],
                        out_hbm.at[cid, jnp.int32(p), pl.ds(sid * RPT2, RPT2)])
        plsc.subcore_barrier()


DVB = 512


def _tc_dinv_body(deg_ref, out_ref):
    tot = lax.dot_general(deg_ref[...], jnp.ones((NW, D), jnp.float32),
                          (((0,), (0,)), ((), ())),
                          preferred_element_type=jnp.float32)
    out_ref[...] = lax.rsqrt(tot + 1.0)


def _tc_dinv(deg_parts):
    return pl.pallas_call(
        _tc_dinv_body,
        grid=(NPAD // DVB,),
        in_specs=[pl.BlockSpec((NW, DVB),
                               lambda i: (jnp.int32(0), jnp.int32(i)))],
        out_specs=pl.BlockSpec((DVB, D),
                               lambda i: (jnp.int32(i), jnp.int32(0))),
        out_shape=jax.ShapeDtypeStruct((NPAD, D), jnp.float32),
    )(deg_parts)


def _tc_first_body(x_ref, w_ref, dinv_ref, hd_ref):
    hd_ref[...] = dinv_ref[...] * jnp.dot(x_ref[...], w_ref[...],
                                          preferred_element_type=jnp.float32)


def _tc_first(x, w1, dinv_b):
    return pl.pallas_call(
        _tc_first_body,
        grid=(NBLK,),
        in_specs=[
            pl.BlockSpec((BLK, D), lambda i: (jnp.int32(i), jnp.int32(0))),
            pl.BlockSpec((D, D), lambda i: (jnp.int32(0), jnp.int32(0))),
            pl.BlockSpec((BLK, D), lambda i: (jnp.int32(i), jnp.int32(0))),
        ],
        out_specs=pl.BlockSpec((BLK, D),
                               lambda i: (jnp.int32(i), jnp.int32(0))),
        out_shape=jax.ShapeDtypeStruct((N, D), jnp.float32),
    )(x, w1, dinv_b)


BPH = SPL // BLK


def _tc_mid_body(s_ref, hd_ref, dinv_ref, b_ref, w_ref, out_ref):
    dinv = dinv_ref[...]
    h = s_ref[0, 0] + s_ref[1, 0] + hd_ref[...]
    h = jnp.maximum(dinv * h + b_ref[...], 0.0)
    out_ref[...] = dinv * jnp.dot(h, w_ref[...],
                                  preferred_element_type=jnp.float32)


def _tc_mid(s, hd, dinv, b2d, w):
    return pl.pallas_call(
        _tc_mid_body,
        grid=(NBLK,),
        in_specs=[
            pl.BlockSpec((NC, 1, BLK, D),
                         lambda i: (jnp.int32(0), jnp.int32(i // BPH),
                                    jnp.int32(i % BPH), jnp.int32(0))),
            pl.BlockSpec((BLK, D), lambda i: (jnp.int32(i), jnp.int32(0))),
            pl.BlockSpec((BLK, D), lambda i: (jnp.int32(i), jnp.int32(0))),
            pl.BlockSpec((1, D), lambda i: (jnp.int32(0), jnp.int32(0))),
            pl.BlockSpec((D, D), lambda i: (jnp.int32(0), jnp.int32(0))),
        ],
        out_specs=pl.BlockSpec((BLK, D),
                               lambda i: (jnp.int32(i), jnp.int32(0))),
        out_shape=jax.ShapeDtypeStruct((N, D), jnp.float32),
    )(s, hd, dinv, b2d, w)


def _tc_pool_body(hd_ref, dinv_ref, batch_ref, wl_ref, bl_ref,
                  out_ref, sums_ref, counts_ref):
    i = pl.program_id(0)

    @pl.when(i == 0)
    def _():
        sums_ref[...] = jnp.zeros_like(sums_ref)
        counts_ref[...] = jnp.zeros_like(counts_ref)

    h = hd_ref[...] / dinv_ref[...]

    gids = lax.broadcasted_iota(jnp.int32, (BLK, G), 1)
    p = (batch_ref[0] == gids).astype(jnp.float32)
    sums_ref[...] += lax.dot_general(
        p, h, (((0,), (0,)), ((), ())), preferred_element_type=jnp.float32)
    counts_ref[...] += lax.dot_general(
        p, jnp.ones((BLK, 1), jnp.float32), (((0,), (0,)), ((), ())),
        preferred_element_type=jnp.float32)

    @pl.when(i == NBLK - 1)
    def _():
        pooled = sums_ref[...] / jnp.maximum(counts_ref[...], 1.0)
        out_ref[...] = jnp.dot(pooled, wl_ref[...],
                               preferred_element_type=jnp.float32) + bl_ref[...]


def _tc_pool(hd, dinv, batch3, wl, bl2d):
    return pl.pallas_call(
        _tc_pool_body,
        grid=(NBLK,),
        in_specs=[
            pl.BlockSpec((BLK, D), lambda i: (jnp.int32(i), jnp.int32(0))),
            pl.BlockSpec((BLK, D), lambda i: (jnp.int32(i), jnp.int32(0))),
            pl.BlockSpec((1, BLK, 1),
                         lambda i: (jnp.int32(i), jnp.int32(0), jnp.int32(0))),
            pl.BlockSpec((G, DOUT), lambda i: (jnp.int32(0), jnp.int32(0))),
            pl.BlockSpec((1, DOUT), lambda i: (jnp.int32(0), jnp.int32(0))),
        ],
        out_specs=pl.BlockSpec((G, DOUT),
                               lambda i: (jnp.int32(0), jnp.int32(0))),
        out_shape=jax.ShapeDtypeStruct((G, DOUT), jnp.float32),
        scratch_shapes=[
            pltpu.VMEM((G, D), jnp.float32),
            pltpu.VMEM((G, 1), jnp.float32),
        ],
    )(hd, dinv, batch3, wl, bl2d)


def kernel(x, edge_index, batch, W1, b1, W2, b2, W3, b3, Wl, bl):
    src = edge_index[0].astype(jnp.int32)
    dst = edge_index[1].astype(jnp.int32)
    pad = EP - E
    packed = src | (dst << 16)
    packed3 = jnp.concatenate(
        [packed, jnp.full((pad,), N << 16, jnp.int32)]).reshape(
            NW, NCH, CHUNK)
    batch3 = batch.astype(jnp.int32).reshape(NBLK, BLK, 1)

    deg_parts = _sc_degree(packed3)
    dinv = _tc_dinv(deg_parts)
    hd1 = _tc_first(x, W1, dinv)

    Ws = jnp.stack([W2, W3, jnp.eye(D, dtype=jnp.float32)])
    bs = jnp.stack([b1.reshape(1, D), b2.reshape(1, D), b3.reshape(1, D)])

    def step(hd, wb):
        w, b = wb
        s = _sc_edge_scatter(hd, packed3)
        return _tc_mid(s, hd, dinv, b, w), None

    hd4, _ = lax.scan(step, hd1, (Ws, bs))

    out = _tc_pool(hd4, dinv, batch3, Wl, bl.reshape(1, DOUT))
    return out
